# Initial kernel scaffold; baseline (speedup 1.0000x reference)
import jax
import jax.numpy as jnp
from jax.experimental import pallas as pl


def kernel(order_feat, embed_table):
    raise NotImplementedError("write your pallas kernel here")


# speedup vs baseline: 5.4454x; 5.4454x over previous
import jax
import jax.numpy as jnp
from jax import lax
from jax.experimental import pallas as pl
from jax.experimental.pallas import tpu as pltpu
from jax.experimental.pallas import tpu_sc as plsc

BATCH = 16384
HIST = 200
OUT_H = 199
D = 32

_info = plsc.get_sparse_core_info()
_NC = _info.num_cores
_NS = _info.num_subcores
_NW = _NC * _NS
_ROWS_PER_W = BATCH // _NW
_RB = 8
_NBLK = _ROWS_PER_W // _RB


def _embed_body(idx_hbm, table_hbm, out_hbm, idx_v, rows_v, sem):
    wid = lax.axis_index("s") * _NC + lax.axis_index("c")
    base = wid * _ROWS_PER_W

    def block(b, carry):
        rbase = base + b * _RB
        pltpu.sync_copy(idx_hbm.at[pl.ds(rbase, _RB)], idx_v)
        handles = []
        for r in range(_RB):
            handles.append(pltpu.async_copy(
                table_hbm.at[idx_v.at[r, pl.ds(0, 128)]],
                rows_v.at[r, pl.ds(0, 128)], sem))
            handles.append(pltpu.async_copy(
                table_hbm.at[idx_v.at[r, pl.ds(128, HIST - 128)]],
                rows_v.at[r, pl.ds(128, HIST - 128)], sem))
        for h in handles:
            h.wait()
        for r in range(_RB):
            pltpu.sync_copy(rows_v.at[r, pl.ds(0, OUT_H)],
                            out_hbm.at[rbase + r])
        return carry

    lax.fori_loop(0, _NBLK, block, 0)


def kernel(order_feat, embed_table):
    k = pl.kernel(
        _embed_body,
        out_type=jax.ShapeDtypeStruct((BATCH, OUT_H, D), jnp.float32),
        mesh=plsc.VectorSubcoreMesh(core_axis_name="c", subcore_axis_name="s"),
        scratch_types=[
            pltpu.VMEM((_RB, HIST), jnp.int32),
            pltpu.VMEM((_RB, HIST, D), jnp.float32),
            pltpu.SemaphoreType.DMA,
        ],
        compiler_params=pltpu.CompilerParams(use_tc_tiling_on_sc=False),
    )
    return k(order_feat, embed_table)


# speedup vs baseline: 5.5802x; 1.0247x over previous
import jax
import jax.numpy as jnp
from jax import lax
from jax.experimental import pallas as pl
from jax.experimental.pallas import tpu as pltpu
from jax.experimental.pallas import tpu_sc as plsc

BATCH = 16384
HIST = 200
OUT_H = 199
D = 32
C1 = 128
C2 = 72

_info = plsc.get_sparse_core_info()
_NC = _info.num_cores
_NS = _info.num_subcores
_NW = _NC * _NS
_ROWS_PER_W = BATCH // _NW
_RB = 8
_NBLK = _ROWS_PER_W // _RB


def _gather_copies(table_hbm, idx_v, rows_v, sem, buf):
    for r in range(_RB):
        yield pltpu.make_async_copy(
            table_hbm.at[idx_v.at[buf, r, pl.ds(0, C1)]],
            rows_v.at[buf, r, pl.ds(0, C1)], sem)
        yield pltpu.make_async_copy(
            table_hbm.at[idx_v.at[buf, r, pl.ds(C1, C2)]],
            rows_v.at[buf, r, pl.ds(C1, C2)], sem)


def _embed_body(idx_hbm, table_hbm, out_hbm,
                idx_v, rows_v, sem_a, sem_b, sem_out):
    wid = lax.axis_index("s") * _NC + lax.axis_index("c")
    base = wid * _ROWS_PER_W
    sems = (sem_a, sem_b)

    def stage_fire(g, buf):
        rbase = base + g * _RB
        pltpu.sync_copy(idx_hbm.at[pl.ds(rbase, _RB)], idx_v.at[buf])
        for cp in _gather_copies(table_hbm, idx_v, rows_v, sems[buf], buf):
            cp.start()

    def drain_gathers(buf):
        for cp in _gather_copies(table_hbm, idx_v, rows_v, sems[buf], buf):
            cp.wait()

    def out_copies(g, buf):
        rbase = base + g * _RB
        for r in range(_RB):
            yield pltpu.make_async_copy(
                rows_v.at[buf, r, pl.ds(0, OUT_H)],
                out_hbm.at[rbase + r], sem_out)

    stage_fire(0, 0)

    def outer(gg, c):
        g0 = 2 * gg
        for j in range(2):
            g = g0 + j
            nxt = 1 - j

            @pl.when(g + 1 < _NBLK)
            def _():
                @pl.when(g >= 1)
                def _():
                    for cp in out_copies(g - 1, nxt):
                        cp.wait()
                stage_fire(g + 1, nxt)

            drain_gathers(j)
            for cp in out_copies(g, j):
                cp.start()
        return c

    lax.fori_loop(0, _NBLK // 2, outer, 0)
    for cp in out_copies(_NBLK - 2, 0):
        cp.wait()
    for cp in out_copies(_NBLK - 1, 1):
        cp.wait()


def kernel(order_feat, embed_table):
    k = pl.kernel(
        _embed_body,
        out_type=jax.ShapeDtypeStruct((BATCH, OUT_H, D), jnp.float32),
        mesh=plsc.VectorSubcoreMesh(core_axis_name="c", subcore_axis_name="s"),
        scratch_types=[
            pltpu.VMEM((2, _RB, HIST), jnp.int32),
            pltpu.VMEM((2, _RB, HIST, D), jnp.float32),
            pltpu.SemaphoreType.DMA,
            pltpu.SemaphoreType.DMA,
            pltpu.SemaphoreType.DMA,
        ],
        compiler_params=pltpu.CompilerParams(use_tc_tiling_on_sc=False),
    )
    return k(order_feat, embed_table)
